# Initial kernel scaffold; baseline (speedup 1.0000x reference)
import jax
import jax.numpy as jnp
from jax.experimental import pallas as pl


def kernel(kg_enc_input, emb, Wl1, Wr1, att1, b1, Wl2, Wr2, att2, b2, pm_W1, pm_b1, pm_W2, pm_b2, lii_W):
    raise NotImplementedError("write your pallas kernel here")


# speedup vs baseline: 17.8381x; 17.8381x over previous
import functools

import jax
import jax.numpy as jnp
from jax import lax
from jax.experimental import pallas as pl
from jax.experimental.pallas import tpu as pltpu
from jax.experimental.pallas import tpu_sc as plsc

V = 10000
D = 128
H = 4
HC = 512
B = 2
T = 2048
BT = B * T
RB = 256
NG = BT // RB
F32 = jnp.float32
PREC = lax.Precision.HIGHEST
V_PAD = 10112


def _sc_gather(emb, ids32):
    mesh = plsc.VectorSubcoreMesh(core_axis_name="c", subcore_axis_name="s")

    @functools.partial(
        pl.kernel,
        out_type=jax.ShapeDtypeStruct((3 * BT, D), F32),
        mesh=mesh,
        scratch_types=[
            pltpu.VMEM((3, 128), jnp.int32),
            pltpu.VMEM((128, D), F32),
            pltpu.SemaphoreType.DMA,
        ],
    )
    def k(emb_hbm, ids_hbm, out_hbm, ibuf, rbuf, sem):
        w = lax.axis_index("s") * 2 + lax.axis_index("c")
        base = pl.multiple_of(w * 384, 128)
        pltpu.sync_copy(ids_hbm.at[w], ibuf)
        for m in range(3):
            pltpu.async_copy(emb_hbm.at[ibuf.at[m]], rbuf, sem).wait()
            pltpu.sync_copy(rbuf, out_hbm.at[pl.ds(base + m * 128, 128)])

    return k(emb, ids32)


def _sc_segsum(dstA, dstB, slots, msgA, msgB, denA, denB):
    mesh = plsc.VectorSubcoreMesh(core_axis_name="c", subcore_axis_name="s")

    @functools.partial(
        pl.kernel,
        out_type=(jax.ShapeDtypeStruct((3, H, BT, D), F32),
                  jax.ShapeDtypeStruct((3, BT, D), F32)),
        mesh=mesh,
        scratch_types=[
            pltpu.VMEM_SHARED((V_PAD, D), F32),
            pltpu.VMEM((128, D), F32),
            pltpu.VMEM((128, D), F32),
            pltpu.VMEM((128, D), F32),
            pltpu.VMEM((2, 128), jnp.int32),
            pltpu.VMEM((1, 128), jnp.int32),
        ],
    )
    def k(dA, dB, sl, mA, mB, dnA, dnB, agg, aggden,
          shared, zbuf, mbuf, gbuf, dbuf, sbuf):
        c = lax.axis_index("c")
        s = lax.axis_index("s")

        def zrow(i, _):
            for j in range(D // 16):
                zbuf[i, pl.ds(j * 16, 16)] = jnp.zeros((16,), F32)
            return 0

        lax.fori_loop(0, 128, zrow, 0)

        def zero_shared():
            base = pl.multiple_of(s * 632, 8)
            for i in range(4):
                pltpu.sync_copy(zbuf, shared.at[pl.ds(base + i * 128, 128)])
            pltpu.sync_copy(zbuf.at[pl.ds(0, 120)],
                            shared.at[pl.ds(base + 512, 120)])

        def scatter_edges(b, srcA, srcB):
            @pl.when(s < 8)
            def _():
                pltpu.sync_copy(dA.at[b, s], dbuf)
                for j in range(2):
                    off = pl.multiple_of(b * T + s * 256 + j * 128, 128)
                    pltpu.sync_copy(srcA.at[pl.ds(off, 128)], mbuf)
                    pltpu.sync_copy(mbuf, shared.at[dbuf.at[j]], add=True)

            @pl.when(s >= 8)
            def _():
                s2 = s - 8
                pltpu.sync_copy(dB.at[b, s2], dbuf)
                for j in range(2):
                    off = pl.multiple_of(b * T + s2 * 256 + j * 128, 128)
                    pltpu.sync_copy(srcB.at[pl.ds(off, 128)], mbuf)
                    pltpu.sync_copy(mbuf, shared.at[dbuf.at[j]], add=True)

        def gather_slots(b, write):
            for j in range(3):
                pltpu.sync_copy(sl.at[j, b, s], sbuf)
                pltpu.sync_copy(shared.at[sbuf.at[0]], gbuf)
                off = pl.multiple_of(b * T + s * 128, 128)
                write(j, off)

        for piece in range(4):
            pid = piece * 2 + c
            b = pid // H
            h = pid - b * H
            zero_shared()
            plsc.subcore_barrier()
            scatter_edges(b, mA.at[h], mB.at[h])
            plsc.subcore_barrier()
            gather_slots(b, lambda j, off: pltpu.sync_copy(
                gbuf, agg.at[j, h, pl.ds(off, 128)]))
            plsc.subcore_barrier()

        b = c
        zero_shared()
        plsc.subcore_barrier()
        scatter_edges(b, dnA, dnB)
        plsc.subcore_barrier()
        gather_slots(b, lambda j, off: pltpu.sync_copy(
            gbuf, aggden.at[j, pl.ds(off, 128)]))

    return k(dstA, dstB, slots, msgA, msgB, denA, denB)


_SCORE_PAIRS = ((0, 1), (1, 2), (0, 0), (1, 1), (2, 2))


def _scores_and_max(xls, xrs, att_ref, e5_ref, cmax_ref):
    es = []
    for (src, dst) in _SCORE_PAIRS:
        for h in range(H):
            m = xls[src][:, 128 * h:128 * (h + 1)] + xrs[dst][:, 128 * h:128 * (h + 1)]
            m = jnp.where(m >= 0, m, 0.2 * m)
            es.append(jnp.sum(m * att_ref[h:h + 1, :], axis=1, keepdims=True))
    e5 = jnp.concatenate(es, axis=1)
    e5_ref[...] = e5
    tmax = jnp.reshape(jnp.max(e5), (1, 1))
    g = pl.program_id(0)

    @pl.when(g == 0)
    def _():
        cmax_ref[...] = tmax

    @pl.when(g > 0)
    def _():
        cmax_ref[...] = jnp.maximum(cmax_ref[...], tmax)


def _a1_body(x_ref, wl_ref, wr_ref, att_ref, xl_ref, e5_ref, cmax_ref):
    xls, xrs = [], []
    for kcol in range(3):
        xk = x_ref[kcol]
        xls.append(jnp.dot(xk, wl_ref[...], preferred_element_type=F32, precision=PREC))
        xrs.append(jnp.dot(xk, wr_ref[...], preferred_element_type=F32, precision=PREC))
        xl_ref[kcol] = xls[kcol]
    _scores_and_max(xls, xrs, att_ref, e5_ref, cmax_ref)


def _b_body(xl_ref, e5_ref, cmax_ref, msgA_ref, msgB_ref, denA_ref, denB_ref,
            smsg_ref, sden_ref):
    cm = cmax_ref[...]
    p = jnp.exp(e5_ref[...] - cm)
    for h in range(H):
        msgA_ref[h] = p[:, h:h + 1] * xl_ref[0][:, 128 * h:128 * (h + 1)]
        msgB_ref[h] = p[:, 4 + h:5 + h] * xl_ref[1][:, 128 * h:128 * (h + 1)]
    z = jnp.zeros((RB, 124), F32)
    denA_ref[...] = jnp.concatenate([p[:, 0:4], z], axis=1)
    denB_ref[...] = jnp.concatenate([p[:, 4:8], z], axis=1)
    for kcol in range(3):
        parts, dparts = [], []
        for h in range(H):
            ps = p[:, 8 + 4 * kcol + h:9 + 4 * kcol + h]
            parts.append(ps * xl_ref[kcol][:, 128 * h:128 * (h + 1)])
            dparts.append(jnp.broadcast_to(ps, (RB, 16)))
        smsg_ref[kcol] = jnp.concatenate(parts, axis=1)
        sden_ref[kcol] = jnp.concatenate(dparts, axis=1)


def _combine(agg_ref, aggden_ref, smsg_ref, sden_ref, bias_ref, kcol):
    num_parts, den_parts = [], []
    for h in range(H):
        num_parts.append(agg_ref[kcol, h])
        den_e = aggden_ref[kcol][:, h:h + 1]
        den_s = sden_ref[kcol][:, 16 * h:16 * h + 1]
        den_parts.append(jnp.broadcast_to(den_e + den_s, (RB, 128)))
    num = jnp.concatenate(num_parts, axis=1)
    den = jnp.concatenate(den_parts, axis=1)
    hout = (num + smsg_ref[kcol]) / (den + 1e-16) + bias_ref[...]
    return jnp.maximum(hout, 0.0)


def _ca2_body(agg_ref, aggden_ref, smsg_ref, sden_ref, bias_ref,
              wl_ref, wr_ref, att_ref, xl_ref, e5_ref, cmax_ref):
    xls, xrs = [], []
    for kcol in range(3):
        xk = _combine(agg_ref, aggden_ref, smsg_ref, sden_ref, bias_ref, kcol)
        xls.append(jnp.dot(xk, wl_ref[...], preferred_element_type=F32, precision=PREC))
        xrs.append(jnp.dot(xk, wr_ref[...], preferred_element_type=F32, precision=PREC))
        xl_ref[kcol] = xls[kcol]
    _scores_and_max(xls, xrs, att_ref, e5_ref, cmax_ref)


def _d_body(agg_ref, aggden_ref, smsg_ref, sden_ref, bias_ref, pw1_ref,
            pb1_ref, pw2_ref, pb2_ref, lii_ref, out_ref):
    ws = []
    for kcol in range(3):
        g = _combine(agg_ref, aggden_ref, smsg_ref, sden_ref, bias_ref, kcol)
        w1 = jnp.dot(g, pw1_ref[...], preferred_element_type=F32, precision=PREC) + pb1_ref[...]
        ws.append(jnp.dot(w1, pw2_ref[...], preferred_element_type=F32, precision=PREC) + pb2_ref[...])
    trip = jnp.concatenate(ws, axis=1)
    out_ref[...] = jnp.dot(trip, lii_ref[...], preferred_element_type=F32, precision=PREC)


def _full(shape):
    return pl.BlockSpec(shape, lambda g: tuple(0 for _ in shape))


def _tc_a1(x, wl, wr, att):
    return pl.pallas_call(
        _a1_body,
        grid=(NG,),
        in_specs=[
            pl.BlockSpec((3, RB, D), lambda g: (0, g, 0)),
            _full((D, HC)),
            _full((D, HC)),
            _full((H, 128)),
        ],
        out_specs=[
            pl.BlockSpec((3, RB, HC), lambda g: (0, g, 0)),
            pl.BlockSpec((RB, 20), lambda g: (g, 0)),
            pl.BlockSpec((1, 1), lambda g: (0, 0)),
        ],
        out_shape=[
            jax.ShapeDtypeStruct((3, BT, HC), F32),
            jax.ShapeDtypeStruct((BT, 20), F32),
            jax.ShapeDtypeStruct((1, 1), F32),
        ],
    )(x, wl, wr, att)


def _tc_b(xl, e5, cmax):
    return pl.pallas_call(
        _b_body,
        grid=(NG,),
        in_specs=[
            pl.BlockSpec((3, RB, HC), lambda g: (0, g, 0)),
            pl.BlockSpec((RB, 20), lambda g: (g, 0)),
            pl.BlockSpec((1, 1), lambda g: (0, 0)),
        ],
        out_specs=[
            pl.BlockSpec((H, RB, D), lambda g: (0, g, 0)),
            pl.BlockSpec((H, RB, D), lambda g: (0, g, 0)),
            pl.BlockSpec((RB, D), lambda g: (g, 0)),
            pl.BlockSpec((RB, D), lambda g: (g, 0)),
            pl.BlockSpec((3, RB, HC), lambda g: (0, g, 0)),
            pl.BlockSpec((3, RB, 64), lambda g: (0, g, 0)),
        ],
        out_shape=[
            jax.ShapeDtypeStruct((H, BT, D), F32),
            jax.ShapeDtypeStruct((H, BT, D), F32),
            jax.ShapeDtypeStruct((BT, D), F32),
            jax.ShapeDtypeStruct((BT, D), F32),
            jax.ShapeDtypeStruct((3, BT, HC), F32),
            jax.ShapeDtypeStruct((3, BT, 64), F32),
        ],
    )(xl, e5, cmax)


def _tc_ca2(agg, aggden, smsg, sden, bias, wl, wr, att):
    return pl.pallas_call(
        _ca2_body,
        grid=(NG,),
        in_specs=[
            pl.BlockSpec((3, H, RB, D), lambda g: (0, 0, g, 0)),
            pl.BlockSpec((3, RB, D), lambda g: (0, g, 0)),
            pl.BlockSpec((3, RB, HC), lambda g: (0, g, 0)),
            pl.BlockSpec((3, RB, 64), lambda g: (0, g, 0)),
            _full((1, HC)),
            _full((HC, HC)),
            _full((HC, HC)),
            _full((H, 128)),
        ],
        out_specs=[
            pl.BlockSpec((3, RB, HC), lambda g: (0, g, 0)),
            pl.BlockSpec((RB, 20), lambda g: (g, 0)),
            pl.BlockSpec((1, 1), lambda g: (0, 0)),
        ],
        out_shape=[
            jax.ShapeDtypeStruct((3, BT, HC), F32),
            jax.ShapeDtypeStruct((BT, 20), F32),
            jax.ShapeDtypeStruct((1, 1), F32),
        ],
    )(agg, aggden, smsg, sden, bias, wl, wr, att)


def _tc_d(agg, aggden, smsg, sden, bias, pw1, pb1, pw2, pb2, lii):
    return pl.pallas_call(
        _d_body,
        grid=(NG,),
        in_specs=[
            pl.BlockSpec((3, H, RB, D), lambda g: (0, 0, g, 0)),
            pl.BlockSpec((3, RB, D), lambda g: (0, g, 0)),
            pl.BlockSpec((3, RB, HC), lambda g: (0, g, 0)),
            pl.BlockSpec((3, RB, 64), lambda g: (0, g, 0)),
            _full((1, HC)),
            _full((HC, D)),
            _full((1, D)),
            _full((D, D)),
            _full((1, D)),
            _full((3 * D, 256)),
        ],
        out_specs=pl.BlockSpec((RB, 256), lambda g: (g, 0)),
        out_shape=jax.ShapeDtypeStruct((BT, 256), F32),
    )(agg, aggden, smsg, sden, bias, pw1, pb1, pw2, pb2, lii)


def kernel(kg_enc_input, emb, Wl1, Wr1, att1, b1, Wl2, Wr2, att2, b2,
           pm_W1, pm_b1, pm_W2, pm_b2, lii_W):
    kg = kg_enc_input.astype(jnp.int32)
    cols = jnp.transpose(kg, (2, 0, 1))
    ids32 = cols.reshape(32, 3, 128)
    dstA = cols[1].reshape(B, 8, 2, 128)
    dstB = cols[2].reshape(B, 8, 2, 128)
    slots = cols.reshape(3, B, T // 128, 1, 128)

    x = _sc_gather(emb, ids32).reshape(3, BT, D)

    xl1, e51, cm1 = _tc_a1(x, Wl1, Wr1, att1)
    msgA1, msgB1, denA1, denB1, smsg1, sden1 = _tc_b(xl1, e51, cm1)
    agg1, aggden1 = _sc_segsum(dstA, dstB, slots, msgA1, msgB1, denA1, denB1)

    xl2, e52, cm2 = _tc_ca2(agg1, aggden1, smsg1, sden1, b1.reshape(1, HC),
                            Wl2, Wr2, att2)
    msgA2, msgB2, denA2, denB2, smsg2, sden2 = _tc_b(xl2, e52, cm2)
    agg2, aggden2 = _sc_segsum(dstA, dstB, slots, msgA2, msgB2, denA2, denB2)

    out = _tc_d(agg2, aggden2, smsg2, sden2, b2.reshape(1, HC),
                pm_W1, pm_b1.reshape(1, D), pm_W2, pm_b2.reshape(1, D), lii_W)
    return out.reshape(B, T, 256)


# speedup vs baseline: 18.7672x; 1.0521x over previous
import functools

import jax
import jax.numpy as jnp
from jax import lax
from jax.experimental import pallas as pl
from jax.experimental.pallas import tpu as pltpu
from jax.experimental.pallas import tpu_sc as plsc

V = 10000
D = 128
H = 4
HC = 512
B = 2
T = 2048
BT = B * T
RB = 256
NG = BT // RB
F32 = jnp.float32
PREC = lax.Precision.HIGHEST
V_PAD = 10112


def _sc_gather(emb, ids32):
    mesh = plsc.VectorSubcoreMesh(core_axis_name="c", subcore_axis_name="s")

    @functools.partial(
        pl.kernel,
        out_type=jax.ShapeDtypeStruct((3 * BT, D), F32),
        mesh=mesh,
        scratch_types=[
            pltpu.VMEM((3, 128), jnp.int32),
            pltpu.VMEM((128, D), F32),
            pltpu.SemaphoreType.DMA,
        ],
    )
    def k(emb_hbm, ids_hbm, out_hbm, ibuf, rbuf, sem):
        w = lax.axis_index("s") * 2 + lax.axis_index("c")
        base = pl.multiple_of(w * 384, 128)
        pltpu.sync_copy(ids_hbm.at[w], ibuf)
        for m in range(3):
            pltpu.async_copy(emb_hbm.at[ibuf.at[m]], rbuf, sem).wait()
            pltpu.sync_copy(rbuf, out_hbm.at[pl.ds(base + m * 128, 128)])

    return k(emb, ids32)


def _sc_segsum(dstA, dstB, slots, msgA, msgB, denA, denB):
    mesh = plsc.VectorSubcoreMesh(core_axis_name="c", subcore_axis_name="s")

    @functools.partial(
        pl.kernel,
        out_type=(jax.ShapeDtypeStruct((3, H, BT, D), F32),
                  jax.ShapeDtypeStruct((3, BT, D), F32)),
        mesh=mesh,
        scratch_types=[
            pltpu.VMEM_SHARED((V_PAD, D), F32),
            pltpu.VMEM((128, D), F32),
            pltpu.VMEM((128, D), F32),
            pltpu.VMEM((128, D), F32),
            pltpu.VMEM((2, 128), jnp.int32),
            pltpu.VMEM((1, 128), jnp.int32),
        ],
    )
    def k(dA, dB, sl, mA, mB, dnA, dnB, agg, aggden,
          shared, zbuf, mbuf, gbuf, dbuf, sbuf):
        c = lax.axis_index("c")
        s = lax.axis_index("s")

        def zrow(i, _):
            for j in range(D // 16):
                zbuf[i, pl.ds(j * 16, 16)] = jnp.zeros((16,), F32)
            return 0

        lax.fori_loop(0, 128, zrow, 0)

        def zero_shared():
            base = pl.multiple_of(s * 632, 8)
            for i in range(4):
                pltpu.sync_copy(zbuf, shared.at[pl.ds(base + i * 128, 128)])
            pltpu.sync_copy(zbuf.at[pl.ds(0, 120)],
                            shared.at[pl.ds(base + 512, 120)])

        def scatter_edges(b, srcA, srcB):
            @pl.when(s < 8)
            def _():
                pltpu.sync_copy(dA.at[b, s], dbuf)
                for j in range(2):
                    off = pl.multiple_of(b * T + s * 256 + j * 128, 128)
                    pltpu.sync_copy(srcA.at[pl.ds(off, 128)], mbuf)
                    pltpu.sync_copy(mbuf, shared.at[dbuf.at[j]], add=True)

            @pl.when(s >= 8)
            def _():
                s2 = s - 8
                pltpu.sync_copy(dB.at[b, s2], dbuf)
                for j in range(2):
                    off = pl.multiple_of(b * T + s2 * 256 + j * 128, 128)
                    pltpu.sync_copy(srcB.at[pl.ds(off, 128)], mbuf)
                    pltpu.sync_copy(mbuf, shared.at[dbuf.at[j]], add=True)

        def gather_slots(b, write):
            for j in range(3):
                pltpu.sync_copy(sl.at[j, b, s], sbuf)
                pltpu.sync_copy(shared.at[sbuf.at[0]], gbuf)
                off = pl.multiple_of(b * T + s * 128, 128)
                write(j, off)

        for piece in range(4):
            pid = piece * 2 + c
            b = pid // H
            h = pid - b * H
            zero_shared()
            plsc.subcore_barrier()
            scatter_edges(b, mA.at[h], mB.at[h])
            plsc.subcore_barrier()
            gather_slots(b, lambda j, off: pltpu.sync_copy(
                gbuf, agg.at[j, h, pl.ds(off, 128)]))
            plsc.subcore_barrier()

        b = c
        zero_shared()
        plsc.subcore_barrier()
        scatter_edges(b, dnA, dnB)
        plsc.subcore_barrier()
        gather_slots(b, lambda j, off: pltpu.sync_copy(
            gbuf, aggden.at[j, pl.ds(off, 128)]))

    return k(dstA, dstB, slots, msgA, msgB, denA, denB)


_SCORE_PAIRS = ((0, 1), (1, 2), (0, 0), (1, 1), (2, 2))


def _scores_and_max(xls, xrs, att_ref, e5_vmem, cm_vmem, g):
    es = []
    for (src, dst) in _SCORE_PAIRS:
        for h in range(H):
            m = xls[src][:, 128 * h:128 * (h + 1)] + xrs[dst][:, 128 * h:128 * (h + 1)]
            m = jnp.where(m >= 0, m, 0.2 * m)
            es.append(jnp.sum(m * att_ref[h:h + 1, :], axis=1, keepdims=True))
    e5 = jnp.concatenate(es, axis=1)
    e5_vmem[pl.ds(g * RB, RB), :] = e5
    tmax = jnp.reshape(jnp.max(e5), (1, 1))

    @pl.when(g == 0)
    def _():
        cm_vmem[...] = tmax

    @pl.when(g > 0)
    def _():
        cm_vmem[...] = jnp.maximum(cm_vmem[...], tmax)


def _msg_phase(xl_vmem, e5_vmem, cm_vmem, g, msgA_ref, msgB_ref,
               denA_ref, denB_ref, smsg_ref, sden_ref):
    cm = cm_vmem[...]
    rows = pl.ds(g * RB, RB)
    p = jnp.exp(e5_vmem[rows, :] - cm)
    xl = [xl_vmem[kcol, rows, :] for kcol in range(3)]
    for h in range(H):
        msgA_ref[h] = p[:, h:h + 1] * xl[0][:, 128 * h:128 * (h + 1)]
        msgB_ref[h] = p[:, 4 + h:5 + h] * xl[1][:, 128 * h:128 * (h + 1)]
    z = jnp.zeros((RB, 124), F32)
    denA_ref[...] = jnp.concatenate([p[:, 0:4], z], axis=1)
    denB_ref[...] = jnp.concatenate([p[:, 4:8], z], axis=1)
    for kcol in range(3):
        parts, dparts = [], []
        for h in range(H):
            ps = p[:, 8 + 4 * kcol + h:9 + 4 * kcol + h]
            parts.append(ps * xl[kcol][:, 128 * h:128 * (h + 1)])
            dparts.append(jnp.broadcast_to(ps, (RB, 16)))
        smsg_ref[kcol] = jnp.concatenate(parts, axis=1)
        sden_ref[kcol] = jnp.concatenate(dparts, axis=1)


def _ab1_body(x_ref, wl_ref, wr_ref, att_ref, msgA_ref, msgB_ref,
              denA_ref, denB_ref, smsg_ref, sden_ref,
              xl_vmem, e5_vmem, cm_vmem):
    ph = pl.program_id(0)
    g = pl.program_id(1)

    @pl.when(ph == 0)
    def _():
        xls, xrs = [], []
        for kcol in range(3):
            xk = x_ref[kcol]
            xls.append(jnp.dot(xk, wl_ref[...], preferred_element_type=F32, precision=PREC))
            xrs.append(jnp.dot(xk, wr_ref[...], preferred_element_type=F32, precision=PREC))
            xl_vmem[kcol, pl.ds(g * RB, RB), :] = xls[kcol]
        _scores_and_max(xls, xrs, att_ref, e5_vmem, cm_vmem, g)

    @pl.when(ph == 1)
    def _():
        _msg_phase(xl_vmem, e5_vmem, cm_vmem, g, msgA_ref, msgB_ref,
                   denA_ref, denB_ref, smsg_ref, sden_ref)


def _combine(agg_ref, aggden_ref, smsg_ref, sden_ref, bias_ref, kcol):
    num_parts, den_parts = [], []
    for h in range(H):
        num_parts.append(agg_ref[kcol, h])
        den_e = aggden_ref[kcol][:, h:h + 1]
        den_s = sden_ref[kcol][:, 16 * h:16 * h + 1]
        den_parts.append(jnp.broadcast_to(den_e + den_s, (RB, 128)))
    num = jnp.concatenate(num_parts, axis=1)
    den = jnp.concatenate(den_parts, axis=1)
    hout = (num + smsg_ref[kcol]) / (den + 1e-16) + bias_ref[...]
    return jnp.maximum(hout, 0.0)


def _cab2_body(agg_ref, aggden_ref, smsg_ref, sden_ref, bias_ref,
               wl_ref, wr_ref, att_ref, msgA_ref, msgB_ref,
               denA_ref, denB_ref, smsg2_ref, sden2_ref,
               xl_vmem, e5_vmem, cm_vmem):
    ph = pl.program_id(0)
    g = pl.program_id(1)

    @pl.when(ph == 0)
    def _():
        xls, xrs = [], []
        for kcol in range(3):
            xk = _combine(agg_ref, aggden_ref, smsg_ref, sden_ref, bias_ref, kcol)
            xls.append(jnp.dot(xk, wl_ref[...], preferred_element_type=F32, precision=PREC))
            xrs.append(jnp.dot(xk, wr_ref[...], preferred_element_type=F32, precision=PREC))
            xl_vmem[kcol, pl.ds(g * RB, RB), :] = xls[kcol]
        _scores_and_max(xls, xrs, att_ref, e5_vmem, cm_vmem, g)

    @pl.when(ph == 1)
    def _():
        _msg_phase(xl_vmem, e5_vmem, cm_vmem, g, msgA_ref, msgB_ref,
                   denA_ref, denB_ref, smsg2_ref, sden2_ref)


def _d_body(agg_ref, aggden_ref, smsg_ref, sden_ref, bias_ref, pw1_ref,
            pb1_ref, pw2_ref, pb2_ref, lii_ref, out_ref):
    ws = []
    for kcol in range(3):
        g = _combine(agg_ref, aggden_ref, smsg_ref, sden_ref, bias_ref, kcol)
        w1 = jnp.dot(g, pw1_ref[...], preferred_element_type=F32, precision=PREC) + pb1_ref[...]
        ws.append(jnp.dot(w1, pw2_ref[...], preferred_element_type=F32, precision=PREC) + pb2_ref[...])
    trip = jnp.concatenate(ws, axis=1)
    out_ref[...] = jnp.dot(trip, lii_ref[...], preferred_element_type=F32, precision=PREC)


def _full(shape):
    return pl.BlockSpec(shape, lambda g: tuple(0 for _ in shape))


def _in_ph0(shape, blk):
    nd = len(shape)
    gdim = nd - 2

    def imap(ph, g):
        gi = jnp.where(ph == 0, g, 0)
        return tuple(gi if i == gdim else 0 for i in range(nd))

    return pl.BlockSpec(blk, imap)


def _out_ph1(blk):
    nd = len(blk)
    gdim = nd - 2

    def imap(ph, g):
        gi = jnp.where(ph == 1, g, 0)
        return tuple(gi if i == gdim else 0 for i in range(nd))

    return pl.BlockSpec(blk, imap)


def _full2(shape):
    return pl.BlockSpec(shape, lambda ph, g: tuple(0 for _ in shape))


_MSG_OUT_SPECS = [
    _out_ph1((H, RB, D)),
    _out_ph1((H, RB, D)),
    _out_ph1((RB, D)),
    _out_ph1((RB, D)),
    _out_ph1((3, RB, HC)),
    _out_ph1((3, RB, 64)),
]

_MSG_OUT_SHAPE = [
    jax.ShapeDtypeStruct((H, BT, D), F32),
    jax.ShapeDtypeStruct((H, BT, D), F32),
    jax.ShapeDtypeStruct((BT, D), F32),
    jax.ShapeDtypeStruct((BT, D), F32),
    jax.ShapeDtypeStruct((3, BT, HC), F32),
    jax.ShapeDtypeStruct((3, BT, 64), F32),
]

_AB_SCRATCH = [
    pltpu.VMEM((3, BT, HC), F32),
    pltpu.VMEM((BT, 20), F32),
    pltpu.VMEM((1, 1), F32),
]


def _tc_ab1(x, wl, wr, att):
    return pl.pallas_call(
        _ab1_body,
        grid=(2, NG),
        in_specs=[
            _in_ph0((3, BT, D), (3, RB, D)),
            _full2((D, HC)),
            _full2((D, HC)),
            _full2((H, 128)),
        ],
        out_specs=_MSG_OUT_SPECS,
        out_shape=_MSG_OUT_SHAPE,
        scratch_shapes=_AB_SCRATCH,
    )(x, wl, wr, att)


def _tc_cab2(agg, aggden, smsg, sden, bias, wl, wr, att):
    return pl.pallas_call(
        _cab2_body,
        grid=(2, NG),
        in_specs=[
            _in_ph0((3, H, BT, D), (3, H, RB, D)),
            _in_ph0((3, BT, D), (3, RB, D)),
            _in_ph0((3, BT, HC), (3, RB, HC)),
            _in_ph0((3, BT, 64), (3, RB, 64)),
            _full2((1, HC)),
            _full2((HC, HC)),
            _full2((HC, HC)),
            _full2((H, 128)),
        ],
        out_specs=_MSG_OUT_SPECS,
        out_shape=_MSG_OUT_SHAPE,
        scratch_shapes=_AB_SCRATCH,
    )(agg, aggden, smsg, sden, bias, wl, wr, att)


def _tc_d(agg, aggden, smsg, sden, bias, pw1, pb1, pw2, pb2, lii):
    return pl.pallas_call(
        _d_body,
        grid=(NG,),
        in_specs=[
            pl.BlockSpec((3, H, RB, D), lambda g: (0, 0, g, 0)),
            pl.BlockSpec((3, RB, D), lambda g: (0, g, 0)),
            pl.BlockSpec((3, RB, HC), lambda g: (0, g, 0)),
            pl.BlockSpec((3, RB, 64), lambda g: (0, g, 0)),
            _full((1, HC)),
            _full((HC, D)),
            _full((1, D)),
            _full((D, D)),
            _full((1, D)),
            _full((3 * D, 256)),
        ],
        out_specs=pl.BlockSpec((RB, 256), lambda g: (g, 0)),
        out_shape=jax.ShapeDtypeStruct((BT, 256), F32),
    )(agg, aggden, smsg, sden, bias, pw1, pb1, pw2, pb2, lii)


def kernel(kg_enc_input, emb, Wl1, Wr1, att1, b1, Wl2, Wr2, att2, b2,
           pm_W1, pm_b1, pm_W2, pm_b2, lii_W):
    kg = kg_enc_input.astype(jnp.int32)
    cols = jnp.transpose(kg, (2, 0, 1))
    ids32 = cols.reshape(32, 3, 128)
    dstA = cols[1].reshape(B, 8, 2, 128)
    dstB = cols[2].reshape(B, 8, 2, 128)
    slots = cols.reshape(3, B, T // 128, 1, 128)

    x = _sc_gather(emb, ids32).reshape(3, BT, D)

    msgA1, msgB1, denA1, denB1, smsg1, sden1 = _tc_ab1(x, Wl1, Wr1, att1)
    agg1, aggden1 = _sc_segsum(dstA, dstB, slots, msgA1, msgB1, denA1, denB1)

    msgA2, msgB2, denA2, denB2, smsg2, sden2 = _tc_cab2(
        agg1, aggden1, smsg1, sden1, b1.reshape(1, HC), Wl2, Wr2, att2)
    agg2, aggden2 = _sc_segsum(dstA, dstB, slots, msgA2, msgB2, denA2, denB2)

    out = _tc_d(agg2, aggden2, smsg2, sden2, b2.reshape(1, HC),
                pm_W1, pm_b1.reshape(1, D), pm_W2, pm_b2.reshape(1, D), lii_W)
    return out.reshape(B, T, 256)


# speedup vs baseline: 23.2592x; 1.2394x over previous
import functools

import jax
import jax.numpy as jnp
from jax import lax
from jax.experimental import pallas as pl
from jax.experimental.pallas import tpu as pltpu
from jax.experimental.pallas import tpu_sc as plsc

V = 10000
D = 128
H = 4
HC = 512
B = 2
T = 2048
BT = B * T
RB = 256
NG = BT // RB
F32 = jnp.float32
V_PAD = 10112


def _sc_gather(emb, ids32):
    mesh = plsc.VectorSubcoreMesh(core_axis_name="c", subcore_axis_name="s")

    @functools.partial(
        pl.kernel,
        out_type=jax.ShapeDtypeStruct((3 * BT, D), F32),
        mesh=mesh,
        scratch_types=[
            pltpu.VMEM((3, 128), jnp.int32),
            pltpu.VMEM((128, D), F32),
            pltpu.SemaphoreType.DMA,
        ],
    )
    def k(emb_hbm, ids_hbm, out_hbm, ibuf, rbuf, sem):
        w = lax.axis_index("s") * 2 + lax.axis_index("c")
        base = pl.multiple_of(w * 384, 128)
        pltpu.sync_copy(ids_hbm.at[w], ibuf)
        for m in range(3):
            pltpu.async_copy(emb_hbm.at[ibuf.at[m]], rbuf, sem).wait()
            pltpu.sync_copy(rbuf, out_hbm.at[pl.ds(base + m * 128, 128)])

    return k(emb, ids32)


def _sc_segsum(dstA, dstB, slots, msgA, msgB, denA, denB):
    mesh = plsc.VectorSubcoreMesh(core_axis_name="c", subcore_axis_name="s")

    @functools.partial(
        pl.kernel,
        out_type=(jax.ShapeDtypeStruct((3, H, BT, D), F32),
                  jax.ShapeDtypeStruct((3, BT, D), F32)),
        mesh=mesh,
        scratch_types=[
            pltpu.VMEM_SHARED((V_PAD, D), F32),
            pltpu.VMEM((128, D), F32),
            pltpu.VMEM((128, D), F32),
            pltpu.VMEM((128, D), F32),
            pltpu.VMEM((2, 128), jnp.int32),
            pltpu.VMEM((1, 128), jnp.int32),
        ],
    )
    def k(dA, dB, sl, mA, mB, dnA, dnB, agg, aggden,
          shared, zbuf, mbuf, gbuf, dbuf, sbuf):
        c = lax.axis_index("c")
        s = lax.axis_index("s")

        def zrow(i, _):
            for j in range(D // 16):
                zbuf[i, pl.ds(j * 16, 16)] = jnp.zeros((16,), F32)
            return 0

        lax.fori_loop(0, 128, zrow, 0)

        def zero_shared():
            base = pl.multiple_of(s * 632, 8)
            for i in range(4):
                pltpu.sync_copy(zbuf, shared.at[pl.ds(base + i * 128, 128)])
            pltpu.sync_copy(zbuf.at[pl.ds(0, 120)],
                            shared.at[pl.ds(base + 512, 120)])

        def scatter_edges(b, srcA, srcB):
            @pl.when(s < 8)
            def _():
                pltpu.sync_copy(dA.at[b, s], dbuf)
                for j in range(2):
                    off = pl.multiple_of(b * T + s * 256 + j * 128, 128)
                    pltpu.sync_copy(srcA.at[pl.ds(off, 128)], mbuf)
                    pltpu.sync_copy(mbuf, shared.at[dbuf.at[j]], add=True)

            @pl.when(s >= 8)
            def _():
                s2 = s - 8
                pltpu.sync_copy(dB.at[b, s2], dbuf)
                for j in range(2):
                    off = pl.multiple_of(b * T + s2 * 256 + j * 128, 128)
                    pltpu.sync_copy(srcB.at[pl.ds(off, 128)], mbuf)
                    pltpu.sync_copy(mbuf, shared.at[dbuf.at[j]], add=True)

        def gather_slots(b, write):
            for j in range(3):
                pltpu.sync_copy(sl.at[j, b, s], sbuf)
                pltpu.sync_copy(shared.at[sbuf.at[0]], gbuf)
                off = pl.multiple_of(b * T + s * 128, 128)
                write(j, off)

        for piece in range(4):
            pid = piece * 2 + c
            b = pid // H
            h = pid - b * H
            zero_shared()
            plsc.subcore_barrier()
            scatter_edges(b, mA.at[h], mB.at[h])
            plsc.subcore_barrier()
            gather_slots(b, lambda j, off: pltpu.sync_copy(
                gbuf, agg.at[j, h, pl.ds(off, 128)]))
            plsc.subcore_barrier()

        b = c
        zero_shared()
        plsc.subcore_barrier()
        scatter_edges(b, dnA, dnB)
        plsc.subcore_barrier()
        gather_slots(b, lambda j, off: pltpu.sync_copy(
            gbuf, aggden.at[j, pl.ds(off, 128)]))

    return k(dstA, dstB, slots, msgA, msgB, denA, denB)


_SCORE_PAIRS = ((0, 1), (1, 2), (0, 0), (1, 1), (2, 2))
BF16 = jnp.bfloat16


def _dot3(a, wh_ref, wlo_ref):
    ah = a.astype(BF16)
    alo = (a - ah.astype(F32)).astype(BF16)
    acc = jnp.dot(ah, wh_ref[...], preferred_element_type=F32)
    acc = acc + jnp.dot(alo, wh_ref[...], preferred_element_type=F32)
    acc = acc + jnp.dot(ah, wlo_ref[...], preferred_element_type=F32)
    return acc


def _scores_and_max(xls, xrs, att_ref, e5_vmem, cm_vmem, g):
    es = []
    for (src, dst) in _SCORE_PAIRS:
        for h in range(H):
            m = xls[src][:, 128 * h:128 * (h + 1)] + xrs[dst][:, 128 * h:128 * (h + 1)]
            m = jnp.where(m >= 0, m, 0.2 * m)
            es.append(jnp.sum(m * att_ref[h:h + 1, :], axis=1, keepdims=True))
    e5 = jnp.concatenate(es, axis=1)
    e5_vmem[pl.ds(g * RB, RB), :] = e5
    tmax = jnp.reshape(jnp.max(e5), (1, 1))

    @pl.when(g == 0)
    def _():
        cm_vmem[...] = tmax

    @pl.when(g > 0)
    def _():
        cm_vmem[...] = jnp.maximum(cm_vmem[...], tmax)


def _msg_phase(xl_vmem, e5_vmem, cm_vmem, g, msgA_ref, msgB_ref,
               denA_ref, denB_ref, smsg_ref, sden_ref):
    cm = cm_vmem[...]
    rows = pl.ds(g * RB, RB)
    p = jnp.exp(e5_vmem[rows, :] - cm)
    xl = [xl_vmem[kcol, rows, :] for kcol in range(3)]
    for h in range(H):
        msgA_ref[h] = p[:, h:h + 1] * xl[0][:, 128 * h:128 * (h + 1)]
        msgB_ref[h] = p[:, 4 + h:5 + h] * xl[1][:, 128 * h:128 * (h + 1)]
    z = jnp.zeros((RB, 124), F32)
    denA_ref[...] = jnp.concatenate([p[:, 0:4], z], axis=1)
    denB_ref[...] = jnp.concatenate([p[:, 4:8], z], axis=1)
    for kcol in range(3):
        parts, dparts = [], []
        for h in range(H):
            ps = p[:, 8 + 4 * kcol + h:9 + 4 * kcol + h]
            parts.append(ps * xl[kcol][:, 128 * h:128 * (h + 1)])
            dparts.append(jnp.broadcast_to(ps, (RB, 16)))
        smsg_ref[kcol] = jnp.concatenate(parts, axis=1)
        sden_ref[kcol] = jnp.concatenate(dparts, axis=1)


def _ab1_body(x_ref, wlh_ref, wll_ref, wrh_ref, wrl_ref, att_ref,
              msgA_ref, msgB_ref, denA_ref, denB_ref, smsg_ref, sden_ref,
              xl_vmem, e5_vmem, cm_vmem):
    ph = pl.program_id(0)
    g = pl.program_id(1)

    @pl.when(ph == 0)
    def _():
        xls, xrs = [], []
        for kcol in range(3):
            xk = x_ref[kcol]
            xls.append(_dot3(xk, wlh_ref, wll_ref))
            xrs.append(_dot3(xk, wrh_ref, wrl_ref))
            xl_vmem[kcol, pl.ds(g * RB, RB), :] = xls[kcol]
        _scores_and_max(xls, xrs, att_ref, e5_vmem, cm_vmem, g)

    @pl.when(ph == 1)
    def _():
        _msg_phase(xl_vmem, e5_vmem, cm_vmem, g, msgA_ref, msgB_ref,
                   denA_ref, denB_ref, smsg_ref, sden_ref)


def _combine(agg_ref, aggden_ref, smsg_ref, sden_ref, bias_ref, kcol):
    num_parts, den_parts = [], []
    for h in range(H):
        num_parts.append(agg_ref[kcol, h])
        den_e = aggden_ref[kcol][:, h:h + 1]
        den_s = sden_ref[kcol][:, 16 * h:16 * h + 1]
        den_parts.append(jnp.broadcast_to(den_e + den_s, (RB, 128)))
    num = jnp.concatenate(num_parts, axis=1)
    den = jnp.concatenate(den_parts, axis=1)
    hout = (num + smsg_ref[kcol]) / (den + 1e-16) + bias_ref[...]
    return jnp.maximum(hout, 0.0)


def _cab2_body(agg_ref, aggden_ref, smsg_ref, sden_ref, bias_ref,
               wlh_ref, wll_ref, wrh_ref, wrl_ref, att_ref, msgA_ref,
               msgB_ref, denA_ref, denB_ref, smsg2_ref, sden2_ref,
               xl_vmem, e5_vmem, cm_vmem):
    ph = pl.program_id(0)
    g = pl.program_id(1)

    @pl.when(ph == 0)
    def _():
        xls, xrs = [], []
        for kcol in range(3):
            xk = _combine(agg_ref, aggden_ref, smsg_ref, sden_ref, bias_ref, kcol)
            xls.append(_dot3(xk, wlh_ref, wll_ref))
            xrs.append(_dot3(xk, wrh_ref, wrl_ref))
            xl_vmem[kcol, pl.ds(g * RB, RB), :] = xls[kcol]
        _scores_and_max(xls, xrs, att_ref, e5_vmem, cm_vmem, g)

    @pl.when(ph == 1)
    def _():
        _msg_phase(xl_vmem, e5_vmem, cm_vmem, g, msgA_ref, msgB_ref,
                   denA_ref, denB_ref, smsg2_ref, sden2_ref)


def _d_body(agg_ref, aggden_ref, smsg_ref, sden_ref, bias_ref,
            pw1h_ref, pw1l_ref, pb1_ref, pw2h_ref, pw2l_ref, pb2_ref,
            liih_ref, liil_ref, out_ref):
    ws = []
    for kcol in range(3):
        g = _combine(agg_ref, aggden_ref, smsg_ref, sden_ref, bias_ref, kcol)
        w1 = _dot3(g, pw1h_ref, pw1l_ref) + pb1_ref[...]
        ws.append(_dot3(w1, pw2h_ref, pw2l_ref) + pb2_ref[...])
    trip = jnp.concatenate(ws, axis=1)
    out_ref[...] = _dot3(trip, liih_ref, liil_ref)


def _full(shape):
    return pl.BlockSpec(shape, lambda g: tuple(0 for _ in shape))


def _in_ph0(shape, blk):
    nd = len(shape)
    gdim = nd - 2

    def imap(ph, g):
        gi = jnp.where(ph == 0, g, 0)
        return tuple(gi if i == gdim else 0 for i in range(nd))

    return pl.BlockSpec(blk, imap)


def _out_ph1(blk):
    nd = len(blk)
    gdim = nd - 2

    def imap(ph, g):
        gi = jnp.where(ph == 1, g, 0)
        return tuple(gi if i == gdim else 0 for i in range(nd))

    return pl.BlockSpec(blk, imap)


def _full2(shape):
    return pl.BlockSpec(shape, lambda ph, g: tuple(0 for _ in shape))


_MSG_OUT_SPECS = [
    _out_ph1((H, RB, D)),
    _out_ph1((H, RB, D)),
    _out_ph1((RB, D)),
    _out_ph1((RB, D)),
    _out_ph1((3, RB, HC)),
    _out_ph1((3, RB, 64)),
]

_MSG_OUT_SHAPE = [
    jax.ShapeDtypeStruct((H, BT, D), F32),
    jax.ShapeDtypeStruct((H, BT, D), F32),
    jax.ShapeDtypeStruct((BT, D), F32),
    jax.ShapeDtypeStruct((BT, D), F32),
    jax.ShapeDtypeStruct((3, BT, HC), F32),
    jax.ShapeDtypeStruct((3, BT, 64), F32),
]

_AB_SCRATCH = [
    pltpu.VMEM((3, BT, HC), F32),
    pltpu.VMEM((BT, 20), F32),
    pltpu.VMEM((1, 1), F32),
]


def _tc_ab1(x, wlh, wll, wrh, wrl, att):
    return pl.pallas_call(
        _ab1_body,
        grid=(2, NG),
        in_specs=[
            _in_ph0((3, BT, D), (3, RB, D)),
            _full2((D, HC)),
            _full2((D, HC)),
            _full2((D, HC)),
            _full2((D, HC)),
            _full2((H, 128)),
        ],
        out_specs=_MSG_OUT_SPECS,
        out_shape=_MSG_OUT_SHAPE,
        scratch_shapes=_AB_SCRATCH,
    )(x, wlh, wll, wrh, wrl, att)


def _tc_cab2(agg, aggden, smsg, sden, bias, wlh, wll, wrh, wrl, att):
    return pl.pallas_call(
        _cab2_body,
        grid=(2, NG),
        in_specs=[
            _in_ph0((3, H, BT, D), (3, H, RB, D)),
            _in_ph0((3, BT, D), (3, RB, D)),
            _in_ph0((3, BT, HC), (3, RB, HC)),
            _in_ph0((3, BT, 64), (3, RB, 64)),
            _full2((1, HC)),
            _full2((HC, HC)),
            _full2((HC, HC)),
            _full2((HC, HC)),
            _full2((HC, HC)),
            _full2((H, 128)),
        ],
        out_specs=_MSG_OUT_SPECS,
        out_shape=_MSG_OUT_SHAPE,
        scratch_shapes=_AB_SCRATCH,
    )(agg, aggden, smsg, sden, bias, wlh, wll, wrh, wrl, att)


def _tc_d(agg, aggden, smsg, sden, bias, pw1h, pw1l, pb1, pw2h, pw2l,
          pb2, liih, liil):
    return pl.pallas_call(
        _d_body,
        grid=(NG,),
        in_specs=[
            pl.BlockSpec((3, H, RB, D), lambda g: (0, 0, g, 0)),
            pl.BlockSpec((3, RB, D), lambda g: (0, g, 0)),
            pl.BlockSpec((3, RB, HC), lambda g: (0, g, 0)),
            pl.BlockSpec((3, RB, 64), lambda g: (0, g, 0)),
            _full((1, HC)),
            _full((HC, D)),
            _full((HC, D)),
            _full((1, D)),
            _full((D, D)),
            _full((D, D)),
            _full((1, D)),
            _full((3 * D, 256)),
            _full((3 * D, 256)),
        ],
        out_specs=pl.BlockSpec((RB, 256), lambda g: (g, 0)),
        out_shape=jax.ShapeDtypeStruct((BT, 256), F32),
    )(agg, aggden, smsg, sden, bias, pw1h, pw1l, pb1, pw2h, pw2l, pb2,
      liih, liil)


def kernel(kg_enc_input, emb, Wl1, Wr1, att1, b1, Wl2, Wr2, att2, b2,
           pm_W1, pm_b1, pm_W2, pm_b2, lii_W):
    kg = kg_enc_input.astype(jnp.int32)
    cols = jnp.transpose(kg, (2, 0, 1))
    ids32 = cols.reshape(32, 3, 128)
    dstA = cols[1].reshape(B, 8, 2, 128)
    dstB = cols[2].reshape(B, 8, 2, 128)
    slots = cols.reshape(3, B, T // 128, 1, 128)

    def split(w):
        wh = w.astype(jnp.bfloat16)
        return wh, (w - wh.astype(F32)).astype(jnp.bfloat16)

    Wl1h, Wl1l = split(Wl1)
    Wr1h, Wr1l = split(Wr1)
    Wl2h, Wl2l = split(Wl2)
    Wr2h, Wr2l = split(Wr2)
    pw1h, pw1l = split(pm_W1)
    pw2h, pw2l = split(pm_W2)
    liih, liil = split(lii_W)

    x = _sc_gather(emb, ids32).reshape(3, BT, D)

    msgA1, msgB1, denA1, denB1, smsg1, sden1 = _tc_ab1(
        x, Wl1h, Wl1l, Wr1h, Wr1l, att1)
    agg1, aggden1 = _sc_segsum(dstA, dstB, slots, msgA1, msgB1, denA1, denB1)

    msgA2, msgB2, denA2, denB2, smsg2, sden2 = _tc_cab2(
        agg1, aggden1, smsg1, sden1, b1.reshape(1, HC),
        Wl2h, Wl2l, Wr2h, Wr2l, att2)
    agg2, aggden2 = _sc_segsum(dstA, dstB, slots, msgA2, msgB2, denA2, denB2)

    out = _tc_d(agg2, aggden2, smsg2, sden2, b2.reshape(1, HC),
                pw1h, pw1l, pm_b1.reshape(1, D), pw2h, pw2l,
                pm_b2.reshape(1, D), liih, liil)
    return out.reshape(B, T, 256)


# speedup vs baseline: 26.0806x; 1.1213x over previous
import functools

import jax
import jax.numpy as jnp
from jax import lax
from jax.experimental import pallas as pl
from jax.experimental.pallas import tpu as pltpu
from jax.experimental.pallas import tpu_sc as plsc

V = 10000
D = 128
H = 4
HC = 512
B = 2
T = 2048
BT = B * T
RB = 256
NG = BT // RB
F32 = jnp.float32
V_PAD = 10112


def _sc_gather(emb, ids32):
    mesh = plsc.VectorSubcoreMesh(core_axis_name="c", subcore_axis_name="s")

    @functools.partial(
        pl.kernel,
        out_type=jax.ShapeDtypeStruct((3 * BT, D), F32),
        mesh=mesh,
        scratch_types=[
            pltpu.VMEM((3, 128), jnp.int32),
            pltpu.VMEM((128, D), F32),
            pltpu.SemaphoreType.DMA,
        ],
    )
    def k(emb_hbm, ids_hbm, out_hbm, ibuf, rbuf, sem):
        w = lax.axis_index("s") * 2 + lax.axis_index("c")
        base = pl.multiple_of(w * 384, 128)
        pltpu.sync_copy(ids_hbm.at[w], ibuf)
        for m in range(3):
            pltpu.async_copy(emb_hbm.at[ibuf.at[m]], rbuf, sem).wait()
            pltpu.sync_copy(rbuf, out_hbm.at[pl.ds(base + m * 128, 128)])

    return k(emb, ids32)


def _sc_segsum(dst, slots, msgs, dens):
    mesh = plsc.VectorSubcoreMesh(core_axis_name="c", subcore_axis_name="s")

    @functools.partial(
        pl.kernel,
        out_type=(jax.ShapeDtypeStruct((3, H, BT, D), F32),
                  jax.ShapeDtypeStruct((3, BT, D), F32)),
        mesh=mesh,
        scratch_types=[
            pltpu.VMEM_SHARED((V_PAD, D), F32),
            pltpu.VMEM((64, D), F32),
            pltpu.VMEM((2, 128, D), F32),
            pltpu.VMEM((2, 2, 128), jnp.int32),
            pltpu.VMEM((2, 3, 1, 128), jnp.int32),
            pltpu.SemaphoreType.DMA,
            pltpu.SemaphoreType.DMA,
            pltpu.SemaphoreType.DMA,
            pltpu.SemaphoreType.DMA,
            pltpu.SemaphoreType.DMA,
        ],
    )
    def k(dst_h, sl, msgs_h, dens_h, agg, aggden,
          shared, zbuf, bufs, dbufs, sbufs,
          semz, sem0, sem1, semg, semp):
        c = lax.axis_index("c")
        s = lax.axis_index("s")
        grp = s // 8
        chunk = s - grp * 8

        def zrow(i, _):
            for j in range(D // 16):
                zbuf[i, pl.ds(j * 16, 16)] = jnp.zeros((16,), F32)
            return 0

        lax.fori_loop(0, 64, zrow, 0)

        cps = []
        for b in range(B):
            cps.append(pltpu.async_copy(dst_h.at[b, s], dbufs.at[b], sem0))
            for j in range(3):
                cps.append(pltpu.async_copy(sl.at[j, b, s], sbufs.at[b, j],
                                            sem1))
        for cp in cps:
            cp.wait()

        def zero_shared():
            base = pl.multiple_of(s * 632, 8)
            zcps = [pltpu.async_copy(
                zbuf, shared.at[pl.ds(base + i * 64, 64)], semz)
                for i in range(9)]
            zcps.append(pltpu.async_copy(
                zbuf.at[pl.ds(0, 56)],
                shared.at[pl.ds(base + 576, 56)], semz))
            for cp in zcps:
                cp.wait()

        def scatter_edges(b, rows_ref):
            sems = [sem0, sem1]
            loads = []
            for j in range(2):
                off = pl.multiple_of(chunk * 256 + j * 128, 128)
                loads.append(pltpu.async_copy(
                    rows_ref.at[pl.ds(off, 128)], bufs.at[j], sems[j]))
            scats = []
            for j in range(2):
                loads[j].wait()
                scats.append(pltpu.async_copy(
                    bufs.at[j], shared.at[dbufs.at[b, j]],
                    sems[j], add=True))
            for cp in scats:
                cp.wait()

        def gather_slots(b, dst_ref):
            off = pl.multiple_of(b * T + s * 128, 128)
            g0 = pltpu.async_copy(shared.at[sbufs.at[b, 0, 0]], bufs.at[0], sem0)
            g1 = pltpu.async_copy(shared.at[sbufs.at[b, 1, 0]], bufs.at[1], sem1)
            g0.wait()
            p0 = pltpu.async_copy(bufs.at[0], dst_ref(0).at[pl.ds(off, 128)], semg)
            g1.wait()
            p1 = pltpu.async_copy(bufs.at[1], dst_ref(1).at[pl.ds(off, 128)], semp)
            p0.wait()
            g2 = pltpu.async_copy(shared.at[sbufs.at[b, 2, 0]], bufs.at[0], sem0)
            g2.wait()
            p2 = pltpu.async_copy(bufs.at[0], dst_ref(2).at[pl.ds(off, 128)], semg)
            return [p1, p2]

        pending = []
        for piece in range(4):
            pid = piece * 2 + c
            b = pid // H
            h = pid - b * H
            zero_shared()
            for cp in pending:
                cp.wait()
            plsc.subcore_barrier()
            scatter_edges(b, msgs_h.at[h, b, grp])
            plsc.subcore_barrier()
            pending = gather_slots(b, lambda j: agg.at[j, h])
            plsc.subcore_barrier()

        zero_shared()
        for cp in pending:
            cp.wait()
        plsc.subcore_barrier()
        scatter_edges(c, dens_h.at[c, grp])
        plsc.subcore_barrier()
        for cp in gather_slots(c, lambda j: aggden.at[j]):
            cp.wait()

    return k(dst, slots, msgs, dens)


_SCORE_PAIRS = ((0, 1), (1, 2), (0, 0), (1, 1), (2, 2))
BF16 = jnp.bfloat16


def _dot3(a, wh_ref, wlo_ref):
    ah = a.astype(BF16)
    alo = (a - ah.astype(F32)).astype(BF16)
    acc = jnp.dot(ah, wh_ref[...], preferred_element_type=F32)
    acc = acc + jnp.dot(alo, wh_ref[...], preferred_element_type=F32)
    acc = acc + jnp.dot(ah, wlo_ref[...], preferred_element_type=F32)
    return acc


def _scores_and_max(xls, xrs, att_ref, e5_vmem, cm_vmem, g):
    es = []
    for (src, dst) in _SCORE_PAIRS:
        for h in range(H):
            m = xls[src][:, 128 * h:128 * (h + 1)] + xrs[dst][:, 128 * h:128 * (h + 1)]
            m = jnp.where(m >= 0, m, 0.2 * m)
            es.append(jnp.sum(m * att_ref[h:h + 1, :], axis=1, keepdims=True))
    e5 = jnp.concatenate(es, axis=1)
    e5_vmem[pl.ds(g * RB, RB), :] = e5
    tmax = jnp.reshape(jnp.max(e5), (1, 1))

    @pl.when(g == 0)
    def _():
        cm_vmem[...] = tmax

    @pl.when(g > 0)
    def _():
        cm_vmem[...] = jnp.maximum(cm_vmem[...], tmax)


def _msg_phase(xl_vmem, e5_vmem, cm_vmem, g, msgs_ref, dens_ref,
               smsg_ref, sden_ref):
    cm = cm_vmem[...]
    rows = pl.ds(g * RB, RB)
    p = jnp.exp(e5_vmem[rows, :] - cm)
    xl = [xl_vmem[kcol, rows, :] for kcol in range(3)]
    for h in range(H):
        msgs_ref[h, 0, 0] = p[:, h:h + 1] * xl[0][:, 128 * h:128 * (h + 1)]
        msgs_ref[h, 0, 1] = p[:, 4 + h:5 + h] * xl[1][:, 128 * h:128 * (h + 1)]
    z = jnp.zeros((RB, 124), F32)
    dens_ref[0, 0] = jnp.concatenate([p[:, 0:4], z], axis=1)
    dens_ref[0, 1] = jnp.concatenate([p[:, 4:8], z], axis=1)
    for kcol in range(3):
        parts, dparts = [], []
        for h in range(H):
            ps = p[:, 8 + 4 * kcol + h:9 + 4 * kcol + h]
            parts.append(ps * xl[kcol][:, 128 * h:128 * (h + 1)])
            dparts.append(jnp.broadcast_to(ps, (RB, 16)))
        smsg_ref[kcol] = jnp.concatenate(parts, axis=1)
        sden_ref[kcol] = jnp.concatenate(dparts, axis=1)


def _ab1_body(x_ref, wlh_ref, wll_ref, wrh_ref, wrl_ref, att_ref,
              msgs_ref, dens_ref, smsg_ref, sden_ref,
              xl_vmem, e5_vmem, cm_vmem):
    ph = pl.program_id(0)
    g = pl.program_id(1)

    @pl.when(ph == 0)
    def _():
        xls, xrs = [], []
        for kcol in range(3):
            xk = x_ref[kcol]
            xls.append(_dot3(xk, wlh_ref, wll_ref))
            xrs.append(_dot3(xk, wrh_ref, wrl_ref))
            xl_vmem[kcol, pl.ds(g * RB, RB), :] = xls[kcol]
        _scores_and_max(xls, xrs, att_ref, e5_vmem, cm_vmem, g)

    @pl.when(ph == 1)
    def _():
        _msg_phase(xl_vmem, e5_vmem, cm_vmem, g, msgs_ref, dens_ref,
                   smsg_ref, sden_ref)


def _combine(agg_ref, aggden_ref, smsg_ref, sden_ref, bias_ref, kcol):
    num_parts, den_parts = [], []
    for h in range(H):
        num_parts.append(agg_ref[kcol, h])
        den_e = aggden_ref[kcol][:, h:h + 1]
        den_s = sden_ref[kcol][:, 16 * h:16 * h + 1]
        den_parts.append(jnp.broadcast_to(den_e + den_s, (RB, 128)))
    num = jnp.concatenate(num_parts, axis=1)
    den = jnp.concatenate(den_parts, axis=1)
    hout = (num + smsg_ref[kcol]) / (den + 1e-16) + bias_ref[...]
    return jnp.maximum(hout, 0.0)


def _cab2_body(agg_ref, aggden_ref, smsg_ref, sden_ref, bias_ref,
               wlh_ref, wll_ref, wrh_ref, wrl_ref, att_ref,
               msgs_ref, dens_ref, smsg2_ref, sden2_ref,
               xl_vmem, e5_vmem, cm_vmem):
    ph = pl.program_id(0)
    g = pl.program_id(1)

    @pl.when(ph == 0)
    def _():
        xls, xrs = [], []
        for kcol in range(3):
            xk = _combine(agg_ref, aggden_ref, smsg_ref, sden_ref, bias_ref, kcol)
            xls.append(_dot3(xk, wlh_ref, wll_ref))
            xrs.append(_dot3(xk, wrh_ref, wrl_ref))
            xl_vmem[kcol, pl.ds(g * RB, RB), :] = xls[kcol]
        _scores_and_max(xls, xrs, att_ref, e5_vmem, cm_vmem, g)

    @pl.when(ph == 1)
    def _():
        _msg_phase(xl_vmem, e5_vmem, cm_vmem, g, msgs_ref, dens_ref,
                   smsg2_ref, sden2_ref)


def _d_body(agg_ref, aggden_ref, smsg_ref, sden_ref, bias_ref,
            pw1h_ref, pw1l_ref, pb1_ref, pw2h_ref, pw2l_ref, pb2_ref,
            liih_ref, liil_ref, out_ref):
    ws = []
    for kcol in range(3):
        g = _combine(agg_ref, aggden_ref, smsg_ref, sden_ref, bias_ref, kcol)
        w1 = _dot3(g, pw1h_ref, pw1l_ref) + pb1_ref[...]
        ws.append(_dot3(w1, pw2h_ref, pw2l_ref) + pb2_ref[...])
    trip = jnp.concatenate(ws, axis=1)
    out_ref[...] = _dot3(trip, liih_ref, liil_ref)


def _full(shape):
    return pl.BlockSpec(shape, lambda g: tuple(0 for _ in shape))


def _in_ph0(shape, blk):
    nd = len(shape)
    gdim = nd - 2

    def imap(ph, g):
        gi = jnp.where(ph == 0, g, 0)
        return tuple(gi if i == gdim else 0 for i in range(nd))

    return pl.BlockSpec(blk, imap)


def _out_ph1(blk):
    nd = len(blk)
    gdim = nd - 2

    def imap(ph, g):
        gi = jnp.where(ph == 1, g, 0)
        return tuple(gi if i == gdim else 0 for i in range(nd))

    return pl.BlockSpec(blk, imap)


def _full2(shape):
    return pl.BlockSpec(shape, lambda ph, g: tuple(0 for _ in shape))


def _msgs_imap(ph, g):
    gi = jnp.where(ph == 1, g, 0)
    return (0, gi // 8, 0, gi % 8, 0)


def _dens_imap(ph, g):
    gi = jnp.where(ph == 1, g, 0)
    return (gi // 8, 0, gi % 8, 0)


_MSG_OUT_SPECS = [
    pl.BlockSpec((H, 1, 2, RB, D), _msgs_imap),
    pl.BlockSpec((1, 2, RB, D), _dens_imap),
    _out_ph1((3, RB, HC)),
    _out_ph1((3, RB, 64)),
]

_MSG_OUT_SHAPE = [
    jax.ShapeDtypeStruct((H, B, 2, T, D), F32),
    jax.ShapeDtypeStruct((B, 2, T, D), F32),
    jax.ShapeDtypeStruct((3, BT, HC), F32),
    jax.ShapeDtypeStruct((3, BT, 64), F32),
]

_AB_SCRATCH = [
    pltpu.VMEM((3, BT, HC), F32),
    pltpu.VMEM((BT, 20), F32),
    pltpu.VMEM((1, 1), F32),
]


def _tc_ab1(x, wlh, wll, wrh, wrl, att):
    return pl.pallas_call(
        _ab1_body,
        grid=(2, NG),
        in_specs=[
            _in_ph0((3, BT, D), (3, RB, D)),
            _full2((D, HC)),
            _full2((D, HC)),
            _full2((D, HC)),
            _full2((D, HC)),
            _full2((H, 128)),
        ],
        out_specs=_MSG_OUT_SPECS,
        out_shape=_MSG_OUT_SHAPE,
        scratch_shapes=_AB_SCRATCH,
    )(x, wlh, wll, wrh, wrl, att)


def _tc_cab2(agg, aggden, smsg, sden, bias, wlh, wll, wrh, wrl, att):
    return pl.pallas_call(
        _cab2_body,
        grid=(2, NG),
        in_specs=[
            _in_ph0((3, H, BT, D), (3, H, RB, D)),
            _in_ph0((3, BT, D), (3, RB, D)),
            _in_ph0((3, BT, HC), (3, RB, HC)),
            _in_ph0((3, BT, 64), (3, RB, 64)),
            _full2((1, HC)),
            _full2((HC, HC)),
            _full2((HC, HC)),
            _full2((HC, HC)),
            _full2((HC, HC)),
            _full2((H, 128)),
        ],
        out_specs=_MSG_OUT_SPECS,
        out_shape=_MSG_OUT_SHAPE,
        scratch_shapes=_AB_SCRATCH,
    )(agg, aggden, smsg, sden, bias, wlh, wll, wrh, wrl, att)


def _tc_d(agg, aggden, smsg, sden, bias, pw1h, pw1l, pb1, pw2h, pw2l,
          pb2, liih, liil):
    return pl.pallas_call(
        _d_body,
        grid=(NG,),
        in_specs=[
            pl.BlockSpec((3, H, RB, D), lambda g: (0, 0, g, 0)),
            pl.BlockSpec((3, RB, D), lambda g: (0, g, 0)),
            pl.BlockSpec((3, RB, HC), lambda g: (0, g, 0)),
            pl.BlockSpec((3, RB, 64), lambda g: (0, g, 0)),
            _full((1, HC)),
            _full((HC, D)),
            _full((HC, D)),
            _full((1, D)),
            _full((D, D)),
            _full((D, D)),
            _full((1, D)),
            _full((3 * D, 256)),
            _full((3 * D, 256)),
        ],
        out_specs=pl.BlockSpec((RB, 256), lambda g: (g, 0)),
        out_shape=jax.ShapeDtypeStruct((BT, 256), F32),
    )(agg, aggden, smsg, sden, bias, pw1h, pw1l, pb1, pw2h, pw2l, pb2,
      liih, liil)


def kernel(kg_enc_input, emb, Wl1, Wr1, att1, b1, Wl2, Wr2, att2, b2,
           pm_W1, pm_b1, pm_W2, pm_b2, lii_W):
    kg = kg_enc_input.astype(jnp.int32)
    cols = jnp.transpose(kg, (2, 0, 1))
    ids32 = cols.reshape(32, 3, 128)
    dst = jnp.concatenate([cols[1].reshape(B, 8, 2, 128),
                           cols[2].reshape(B, 8, 2, 128)], axis=1)
    slots = cols.reshape(3, B, T // 128, 1, 128)

    def split(w):
        wh = w.astype(jnp.bfloat16)
        return wh, (w - wh.astype(F32)).astype(jnp.bfloat16)

    Wl1h, Wl1l = split(Wl1)
    Wr1h, Wr1l = split(Wr1)
    Wl2h, Wl2l = split(Wl2)
    Wr2h, Wr2l = split(Wr2)
    pw1h, pw1l = split(pm_W1)
    pw2h, pw2l = split(pm_W2)
    liih, liil = split(lii_W)

    x = _sc_gather(emb, ids32).reshape(3, BT, D)

    msgs1, dens1, smsg1, sden1 = _tc_ab1(x, Wl1h, Wl1l, Wr1h, Wr1l, att1)
    agg1, aggden1 = _sc_segsum(dst, slots, msgs1, dens1)

    msgs2, dens2, smsg2, sden2 = _tc_cab2(
        agg1, aggden1, smsg1, sden1, b1.reshape(1, HC),
        Wl2h, Wl2l, Wr2h, Wr2l, att2)
    agg2, aggden2 = _sc_segsum(dst, slots, msgs2, dens2)

    out = _tc_d(agg2, aggden2, smsg2, sden2, b2.reshape(1, HC),
                pw1h, pw1l, pm_b1.reshape(1, D), pw2h, pw2l,
                pm_b2.reshape(1, D), liih, liil)
    return out.reshape(B, T, 256)
